# baseline (speedup 1.0000x reference)
import jax
import jax.numpy as jnp
from jax import lax
from jax.experimental import pallas as pl
from jax.experimental.pallas import tpu as pltpu
from jax.experimental.pallas import tpu_sc as plsc

B = 4096
D = 512
PRE = 7
SH = 8
ATT = 16
SUF = 46
ROWS = PRE + SH + ATT + SUF
TPAD = 64
HEAD = PRE + SH + 1
SUFB = SUF + 2
NC = 2
NS = 16
NW = NC * NS
CHUNK = B // NW


def _body(attr_hbm, vids_hbm, tmpl_hbm, out_hbm, tmpl_v, abuf_v, vids_v):
    c = lax.axis_index("c")
    s = lax.axis_index("s")
    wid = s * NC + c
    base = wid * CHUNK

    pltpu.sync_copy(tmpl_hbm, tmpl_v)
    pltpu.sync_copy(vids_hbm.at[pl.ds(base, CHUNK)], vids_v)
    lanes = lax.iota(jnp.int32, 16)

    def elem(i, carry):
        g = pl.multiple_of((i // 16) * 16, 16)
        lane = i - g
        chunk16 = vids_v[pl.ds(g, 16)]
        vid = jnp.sum(jnp.where(lanes == lane, chunk16, 0))
        off = pl.multiple_of(vid * TPAD, 8)
        b = base + i
        ob = b * ROWS
        pltpu.sync_copy(tmpl_v.at[pl.ds(off, HEAD)],
                        out_hbm.at[pl.ds(ob, HEAD)])
        pltpu.sync_copy(tmpl_v.at[pl.ds(off + HEAD, SUFB)],
                        out_hbm.at[pl.ds(ob + ROWS - SUFB, SUFB)])
        pltpu.sync_copy(attr_hbm.at[pl.ds(b * ATT, ATT)], abuf_v)
        pltpu.sync_copy(abuf_v, out_hbm.at[pl.ds(ob + PRE + SH, ATT)])
        return carry

    lax.fori_loop(0, CHUNK, elem, 0)


def kernel(attribute, viewids, share_vectors, token_prefix, token_suffix):
    attr2 = attribute.reshape(B * ATT, D)
    vids = viewids.astype(jnp.int32)
    tp2 = token_prefix.reshape(2, PRE, D)
    ts2 = token_suffix.reshape(2, SUF, D)
    sh2 = jnp.broadcast_to(share_vectors[None], (2, SH, D))
    pad = jnp.zeros((2, 3, D), jnp.float32)
    tmpl = jnp.concatenate([tp2, sh2, pad, ts2], axis=1).reshape(2 * TPAD, D)

    out_flat = pl.kernel(
        _body,
        out_type=jax.ShapeDtypeStruct((B * ROWS, D), jnp.float32),
        mesh=plsc.VectorSubcoreMesh(core_axis_name="c", subcore_axis_name="s"),
        compiler_params=pltpu.CompilerParams(use_tc_tiling_on_sc=False,
                                             needs_layout_passes=False),
        scratch_types=[
            pltpu.VMEM((2 * TPAD, D), jnp.float32),
            pltpu.VMEM((ATT, D), jnp.float32),
            pltpu.VMEM((CHUNK,), jnp.int32),
        ],
    )(attr2, vids, tmpl)
    return out_flat.reshape(B, ROWS, D)


# speedup vs baseline: 1.0572x; 1.0572x over previous
import jax
import jax.numpy as jnp
from jax import lax
from jax.experimental import pallas as pl
from jax.experimental.pallas import tpu as pltpu
from jax.experimental.pallas import tpu_sc as plsc

B = 4096
D = 512
PRE = 7
SH = 8
ATT = 16
SUF = 46
ROWS = PRE + SH + ATT + SUF
HEAD = PRE + SH
TROWS = HEAD + SUF
NC = 2
NS = 16
NW = NC * NS
CHUNK = B // NW
NB = 4
PF = 2
LAG = 4
NQ = CHUNK // NB


def _body(attr_hbm, vids_hbm, tmpl_hbm, out_hbm, tmpl_v, vids_v,
          ab0, ab1, ab2, ab3, sem_t, sg0, sg1, sg2, sg3, sw0, sw1, sw2, sw3):
    abufs = (ab0, ab1, ab2, ab3)
    sgs = (sg0, sg1, sg2, sg3)
    sws = (sw0, sw1, sw2, sw3)

    c = lax.axis_index("c")
    s = lax.axis_index("s")
    wid = s * NC + c
    base = wid * CHUNK

    pltpu.sync_copy(tmpl_hbm, tmpl_v)
    pltpu.sync_copy(vids_hbm.at[pl.ds(base, CHUNK)], vids_v)
    lanes = lax.iota(jnp.int32, 16)

    def attr_src(i):
        return attr_hbm.at[pl.ds((base + i) * ATT, ATT)]

    def wait_tmpl_writes(n=1):
        for _ in range(n):
            pltpu.make_async_copy(tmpl_v.at[pl.ds(0, HEAD)],
                                  out_hbm.at[pl.ds(0, HEAD)], sem_t).wait()
            pltpu.make_async_copy(tmpl_v.at[pl.ds(0, SUF)],
                                  out_hbm.at[pl.ds(0, SUF)], sem_t).wait()

    def wait_attr_write(slot):
        pltpu.make_async_copy(abufs[slot], out_hbm.at[pl.ds(0, ATT)],
                              sws[slot]).wait()

    def estep(i, slot, tmpl_lag, wait_prev, gather_next):
        b = base + i
        ob = b * ROWS
        pltpu.make_async_copy(attr_src(i), abufs[slot], sgs[slot]).wait()
        grp = pl.multiple_of((i // 16) * 16, 16)
        chunk16 = vids_v[pl.ds(grp, 16)]
        vid = jnp.sum(jnp.where(lanes == i - grp, chunk16, 0))
        off = vid * TROWS
        pltpu.async_copy(tmpl_v.at[pl.ds(off, HEAD)],
                         out_hbm.at[pl.ds(ob, HEAD)], sem_t)
        pltpu.async_copy(tmpl_v.at[pl.ds(off + HEAD, SUF)],
                         out_hbm.at[pl.ds(ob + HEAD + ATT, SUF)], sem_t)
        if tmpl_lag:
            wait_tmpl_writes(1)
        pltpu.async_copy(abufs[slot],
                         out_hbm.at[pl.ds(ob + HEAD, ATT)], sws[slot])
        if gather_next:
            ns = (slot + PF) % NB
            if wait_prev:
                wait_attr_write(ns)
            pltpu.async_copy(attr_src(i + PF), abufs[ns], sgs[ns])

    pltpu.async_copy(attr_src(0), ab0, sg0)
    pltpu.async_copy(attr_src(1), ab1, sg1)
    estep(0, 0, False, False, True)
    estep(1, 1, False, False, True)
    estep(2, 2, False, True, True)
    estep(3, 3, False, True, True)

    def quad(q, carry):
        for slot in range(NB):
            estep(q * NB + slot, slot, True, True, True)
        return carry

    lax.fori_loop(1, NQ - 1, quad, 0)

    estep(CHUNK - 4, 0, True, True, True)
    estep(CHUNK - 3, 1, True, True, True)
    estep(CHUNK - 2, 2, True, False, False)
    estep(CHUNK - 1, 3, True, False, False)
    wait_tmpl_writes(LAG)
    for slot in range(NB):
        wait_attr_write(slot)


def kernel(attribute, viewids, share_vectors, token_prefix, token_suffix):
    attr2 = attribute.reshape(B * ATT, D)
    vids = viewids.astype(jnp.int32)
    tp2 = token_prefix.reshape(2, PRE, D)
    ts2 = token_suffix.reshape(2, SUF, D)
    sh2 = jnp.broadcast_to(share_vectors[None], (2, SH, D))
    tmpl = jnp.concatenate([tp2, sh2, ts2], axis=1).reshape(2 * TROWS, D)

    out_flat = pl.kernel(
        _body,
        out_type=jax.ShapeDtypeStruct((B * ROWS, D), jnp.float32),
        mesh=plsc.VectorSubcoreMesh(core_axis_name="c", subcore_axis_name="s"),
        compiler_params=pltpu.CompilerParams(use_tc_tiling_on_sc=False,
                                             needs_layout_passes=False),
        scratch_types=[
            pltpu.VMEM((2 * TROWS, D), jnp.float32),
            pltpu.VMEM((CHUNK,), jnp.int32),
            pltpu.VMEM((ATT, D), jnp.float32),
            pltpu.VMEM((ATT, D), jnp.float32),
            pltpu.VMEM((ATT, D), jnp.float32),
            pltpu.VMEM((ATT, D), jnp.float32),
            pltpu.SemaphoreType.DMA,
            pltpu.SemaphoreType.DMA,
            pltpu.SemaphoreType.DMA,
            pltpu.SemaphoreType.DMA,
            pltpu.SemaphoreType.DMA,
            pltpu.SemaphoreType.DMA,
            pltpu.SemaphoreType.DMA,
            pltpu.SemaphoreType.DMA,
            pltpu.SemaphoreType.DMA,
        ],
    )(attr2, vids, tmpl)
    return out_flat.reshape(B, ROWS, D)


# speedup vs baseline: 1.5637x; 1.4791x over previous
import jax
import jax.numpy as jnp
from jax import lax
from jax.experimental import pallas as pl
from jax.experimental.pallas import tpu as pltpu
from jax.experimental.pallas import tpu_sc as plsc

B = 4096
D = 512
PRE = 7
SH = 8
ATT = 16
SUF = 46
ROWS = PRE + SH + ATT + SUF
NC = 2
NS = 16
NW = NC * NS
CHUNK = B // NW

TP = 69


def _body(attr_hbm, vids_hbm, tmpl_hbm, out_hbm,
          tmpl_v, vids_v, sbuf, b1, b2):
    c = lax.axis_index("c")
    s = lax.axis_index("s")
    wid = s * NC + c
    base = wid * CHUNK

    pltpu.sync_copy(tmpl_hbm, tmpl_v)
    pltpu.sync_copy(vids_hbm.at[pl.ds(base, CHUNK)], vids_v)
    lanes = lax.iota(jnp.int32, 16)

    def elem(i, carry):
        b = base + i
        grp = pl.multiple_of((i // 16) * 16, 16)
        chunk16 = vids_v[pl.ds(grp, 16)]
        vid = jnp.sum(jnp.where(lanes == i - grp, chunk16, 0))
        page = out_hbm.at[b]
        tpl = tmpl_v.at[vid]

        pltpu.sync_copy(attr_hbm.at[pl.ds(b * ATT, ATT)], sbuf)

        pltpu.sync_copy(tpl.at[pl.ds(0, 8)], page.at[pl.ds(0, 8)])
        pltpu.sync_copy(tpl.at[pl.ds(16, 40)], page.at[pl.ds(32, 40)])
        pltpu.sync_copy(tpl.at[pl.ds(64, 5)], page.at[pl.ds(72, 5)])

        for r in range(7):
            for cc in range(0, D, 16):
                b1[r, pl.ds(cc, 16)] = tmpl_v[vid, 8 + r, pl.ds(cc, 16)]
        for cc in range(0, D, 16):
            b1[7, pl.ds(cc, 16)] = sbuf[0, pl.ds(cc, 16)]
        pltpu.sync_copy(b1, page.at[pl.ds(8, 8)])

        for r in range(15):
            for cc in range(0, D, 16):
                b2[r, pl.ds(cc, 16)] = sbuf[r + 1, pl.ds(cc, 16)]
        for cc in range(0, D, 16):
            b2[15, pl.ds(cc, 16)] = tmpl_v[vid, 56, pl.ds(cc, 16)]
        pltpu.sync_copy(b2, page.at[pl.ds(16, 16)])
        return carry

    lax.fori_loop(0, CHUNK, elem, 0)


def kernel(attribute, viewids, share_vectors, token_prefix, token_suffix):
    attr2 = attribute.reshape(B * ATT, D)
    vids = viewids.astype(jnp.int32)
    tp2 = token_prefix.reshape(2, PRE, D)
    ts2 = token_suffix.reshape(2, SUF, D)
    sh2 = jnp.broadcast_to(share_vectors[None], (2, SH, D))
    z = jnp.zeros((2, 1, D), jnp.float32)
    tmpl = jnp.concatenate([
        tp2, sh2[:, :1],
        sh2[:, 1:], z,
        ts2[:, 1:41],
        ts2[:, :1],
        jnp.zeros((2, 7, D), jnp.float32),
        ts2[:, 41:],
    ], axis=1)

    out = pl.kernel(
        _body,
        out_type=jax.ShapeDtypeStruct((B, ROWS, D), jnp.float32),
        mesh=plsc.VectorSubcoreMesh(core_axis_name="c", subcore_axis_name="s"),
        compiler_params=pltpu.CompilerParams(needs_layout_passes=False),
        scratch_types=[
            pltpu.VMEM((2, TP, D), jnp.float32),
            pltpu.VMEM((CHUNK,), jnp.int32),
            pltpu.VMEM((ATT, D), jnp.float32),
            pltpu.VMEM((8, D), jnp.float32),
            pltpu.VMEM((ATT, D), jnp.float32),
        ],
    )(attr2, vids, tmpl)
    return out


# speedup vs baseline: 2.0265x; 1.2960x over previous
import jax
import jax.numpy as jnp
from jax import lax
from jax.experimental import pallas as pl
from jax.experimental.pallas import tpu as pltpu
from jax.experimental.pallas import tpu_sc as plsc

B = 4096
D = 512
PRE = 7
SH = 8
ATT = 16
SUF = 46
ROWS = PRE + SH + ATT + SUF
NC = 2
NS = 16
NW = NC * NS
CHUNK = B // NW

TP = 69


def _body(attr_hbm, vids_hbm, tmpl_hbm, out_hbm, tmpl_v, vids_v,
          sb0, sb1, b10, b11, b20, b21, sem_t, sg0, sg1, sw0, sw1):
    sbufs = (sb0, sb1)
    b1s = (b10, b11)
    b2s = (b20, b21)
    sgs = (sg0, sg1)
    sws = (sw0, sw1)

    c = lax.axis_index("c")
    s = lax.axis_index("s")
    wid = s * NC + c
    base = wid * CHUNK

    pltpu.sync_copy(tmpl_hbm, tmpl_v)
    pltpu.sync_copy(vids_hbm.at[pl.ds(base, CHUNK)], vids_v)
    lanes = lax.iota(jnp.int32, 16)

    def attr_src(i):
        return attr_hbm.at[pl.ds((base + i) * ATT, ATT)]

    def wait_slabs(slot):
        pltpu.make_async_copy(tmpl_v.at[0, pl.ds(0, 8)],
                              out_hbm.at[0, pl.ds(0, 8)], sem_t).wait()
        pltpu.make_async_copy(tmpl_v.at[0, pl.ds(16, 40)],
                              out_hbm.at[0, pl.ds(32, 40)], sem_t).wait()
        pltpu.make_async_copy(tmpl_v.at[0, pl.ds(64, 5)],
                              out_hbm.at[0, pl.ds(72, 5)], sem_t).wait()

    def wait_bwrites(slot):
        pltpu.make_async_copy(b1s[slot], out_hbm.at[0, pl.ds(8, 8)],
                              sws[slot]).wait()
        pltpu.make_async_copy(b2s[slot], out_hbm.at[0, pl.ds(16, 16)],
                              sws[slot]).wait()

    def estep(i, slot, steady, tail):
        b = base + i
        page = out_hbm.at[b]
        if steady:
            wait_bwrites(slot)
        pltpu.make_async_copy(attr_src(i), sbufs[slot], sgs[slot]).wait()
        sbuf = sbufs[slot]
        b1 = b1s[slot]
        b2 = b2s[slot]
        grp = pl.multiple_of((i // 16) * 16, 16)
        chunk16 = vids_v[pl.ds(grp, 16)]
        vid = jnp.sum(jnp.where(lanes == i - grp, chunk16, 0))
        tpl = tmpl_v.at[vid]

        for r in range(7):
            for cc in range(0, D, 16):
                b1[r, pl.ds(cc, 16)] = tmpl_v[vid, 8 + r, pl.ds(cc, 16)]
        for cc in range(0, D, 16):
            b1[7, pl.ds(cc, 16)] = sbuf[0, pl.ds(cc, 16)]
        for r in range(15):
            for cc in range(0, D, 16):
                b2[r, pl.ds(cc, 16)] = sbuf[r + 1, pl.ds(cc, 16)]
        for cc in range(0, D, 16):
            b2[15, pl.ds(cc, 16)] = tmpl_v[vid, 56, pl.ds(cc, 16)]

        if not tail:
            pltpu.async_copy(attr_src(i + 2), sbufs[slot], sgs[slot])
        pltpu.async_copy(tpl.at[pl.ds(0, 8)], page.at[pl.ds(0, 8)], sem_t)
        pltpu.async_copy(tpl.at[pl.ds(16, 40)], page.at[pl.ds(32, 40)], sem_t)
        pltpu.async_copy(tpl.at[pl.ds(64, 5)], page.at[pl.ds(72, 5)], sem_t)
        pltpu.async_copy(b1, page.at[pl.ds(8, 8)], sws[slot])
        pltpu.async_copy(b2, page.at[pl.ds(16, 16)], sws[slot])
        if steady:
            wait_slabs(slot)

    pltpu.async_copy(attr_src(0), sb0, sg0)
    pltpu.async_copy(attr_src(1), sb1, sg1)
    estep(0, 0, False, False)
    estep(1, 1, False, False)

    def pair(q, carry):
        estep(2 * q, 0, True, False)
        estep(2 * q + 1, 1, True, False)
        return carry

    lax.fori_loop(1, CHUNK // 2 - 1, pair, 0)

    estep(CHUNK - 2, 0, True, True)
    estep(CHUNK - 1, 1, True, True)
    for slot in range(2):
        wait_bwrites(slot)
        wait_slabs(slot)


def kernel(attribute, viewids, share_vectors, token_prefix, token_suffix):
    attr2 = attribute.reshape(B * ATT, D)
    vids = viewids.astype(jnp.int32)
    tp2 = token_prefix.reshape(2, PRE, D)
    ts2 = token_suffix.reshape(2, SUF, D)
    sh2 = jnp.broadcast_to(share_vectors[None], (2, SH, D))
    z = jnp.zeros((2, 1, D), jnp.float32)
    tmpl = jnp.concatenate([
        tp2, sh2[:, :1],
        sh2[:, 1:], z,
        ts2[:, 1:41],
        ts2[:, :1],
        jnp.zeros((2, 7, D), jnp.float32),
        ts2[:, 41:],
    ], axis=1)

    out = pl.kernel(
        _body,
        out_type=jax.ShapeDtypeStruct((B, ROWS, D), jnp.float32),
        mesh=plsc.VectorSubcoreMesh(core_axis_name="c", subcore_axis_name="s"),
        compiler_params=pltpu.CompilerParams(needs_layout_passes=False),
        scratch_types=[
            pltpu.VMEM((2, TP, D), jnp.float32),
            pltpu.VMEM((CHUNK,), jnp.int32),
            pltpu.VMEM((ATT, D), jnp.float32),
            pltpu.VMEM((ATT, D), jnp.float32),
            pltpu.VMEM((8, D), jnp.float32),
            pltpu.VMEM((8, D), jnp.float32),
            pltpu.VMEM((ATT, D), jnp.float32),
            pltpu.VMEM((ATT, D), jnp.float32),
            pltpu.SemaphoreType.DMA,
            pltpu.SemaphoreType.DMA,
            pltpu.SemaphoreType.DMA,
            pltpu.SemaphoreType.DMA,
            pltpu.SemaphoreType.DMA,
        ],
    )(attr2, vids, tmpl)
    return out


# speedup vs baseline: 4.1625x; 2.0541x over previous
import jax
import jax.numpy as jnp
from jax import lax
from jax.experimental import pallas as pl
from jax.experimental.pallas import tpu as pltpu
from jax.experimental.pallas import tpu_sc as plsc

B = 4096
D = 512
PRE = 7
SH = 8
ATT = 16
SUF = 46
ROWS = PRE + SH + ATT + SUF
TQ = PRE + SH + SUF
NC = 2
NS = 16
NW = NC * NS
CHUNK = B // NW
HB = 32
NH = CHUNK // HB


def _body(attr_hbm, vids_hbm, tt_hbm, out_hbm,
          tt_v, vids_v, sl0, sl1, idxb, sw0, sw1, sg0, sg1):
    slabs = (sl0, sl1)
    sws = (sw0, sw1)
    sgs = (sg0, sg1)

    c = lax.axis_index("c")
    s = lax.axis_index("s")
    wid = s * NC + c
    base = wid * CHUNK

    pltpu.sync_copy(tt_hbm, tt_v)
    pltpu.sync_copy(vids_hbm.at[pl.ds(base, CHUNK)], vids_v)
    lanes = lax.iota(jnp.int32, 16)

    def wait_w(slot, n=1):
        for _ in range(n):
            pltpu.make_async_copy(slabs[slot], out_hbm.at[0, pl.ds(0, HB)],
                                  sws[slot]).wait()

    def assemble_half(q, slot, jb):
        slab = slabs[slot]
        for ch in range(2):
            co = ch * 256
            t0 = [tt_v[0, q, pl.ds(co + cc * 16, 16)] for cc in range(16)]
            t1 = [tt_v[1, q, pl.ds(co + cc * 16, 16)] for cc in range(16)]

            def row(j2, carry):
                for u in range(2):
                    j = j2 * 2 + u
                    vidv = plsc.load_gather(
                        vids_v, [jnp.full((16,), jb + j, jnp.int32)])
                    m = vidv != 0
                    for cc in range(16):
                        slab[j, pl.ds(co + cc * 16, 16)] = (
                            jnp.where(m, t1[cc], t0[cc]))
                return carry

            lax.fori_loop(0, HB // 2, row, 0)

    for si in range(SH):
        q = PRE + si
        slot = si % 2
        if si >= 2:
            wait_w(slot, NH)
        slab = slabs[slot]
        for ch in range(2):
            co = ch * 256
            t = [tt_v[0, q, pl.ds(co + cc * 16, 16)] for cc in range(16)]

            def srow(j, carry):
                for cc in range(16):
                    slab[j, pl.ds(co + cc * 16, 16)] = t[cc]
                return carry

            lax.fori_loop(0, HB, srow, 0)
        for h in range(NH):
            pltpu.async_copy(slab, out_hbm.at[q, pl.ds(base + h * HB, HB)],
                             sws[slot])

    def tplane(q, first):
        r = q + jnp.where(q >= PRE + SH, ATT, 0)
        for h in range(NH):
            slot = h % 2
            wait_w(slot, NH if (first and h < 2) else 1)
            assemble_half(q, slot, h * HB)
            pltpu.async_copy(slabs[slot],
                             out_hbm.at[r, pl.ds(base + h * HB, HB)],
                             sws[slot])

    tplane(jnp.int32(0), True)

    def tloop(i, carry):
        q = i + jnp.where(i >= PRE, SH, 0)
        tplane(q, False)
        return carry

    lax.fori_loop(1, PRE + SUF, tloop, 0)

    def aplane(k, carry):
        r = PRE + SH + k
        for h in range(NH):
            for cc in range(2):
                li = lanes + (h * HB + cc * 16)
                idxb[h, pl.ds(cc * 16, 16)] = (base + li) * ATT + k
        for h in range(NH):
            slot = h % 2
            wait_w(slot)
            pltpu.async_copy(attr_hbm.at[idxb.at[h]], slabs[slot],
                             sgs[slot]).wait()
            pltpu.async_copy(slabs[slot],
                             out_hbm.at[r, pl.ds(base + h * HB, HB)],
                             sws[slot])
        return carry

    lax.fori_loop(0, ATT, aplane, 0)

    wait_w(0)
    wait_w(1)


def kernel(attribute, viewids, share_vectors, token_prefix, token_suffix):
    attr2 = attribute.reshape(B * ATT, D)
    vids = viewids.astype(jnp.int32)
    tp2 = token_prefix.reshape(2, PRE, D)
    ts2 = token_suffix.reshape(2, SUF, D)
    sh2 = jnp.broadcast_to(share_vectors[None], (2, SH, D))
    tt = jnp.concatenate([tp2, sh2, ts2], axis=1)

    out2 = pl.kernel(
        _body,
        out_type=jax.ShapeDtypeStruct((ROWS, B, D), jnp.float32),
        mesh=plsc.VectorSubcoreMesh(core_axis_name="c", subcore_axis_name="s"),
        compiler_params=pltpu.CompilerParams(needs_layout_passes=False),
        scratch_types=[
            pltpu.VMEM((2, TQ, D), jnp.float32),
            pltpu.VMEM((CHUNK,), jnp.int32),
            pltpu.VMEM((HB, D), jnp.float32),
            pltpu.VMEM((HB, D), jnp.float32),
            pltpu.VMEM((NH, HB), jnp.int32),
            pltpu.SemaphoreType.DMA,
            pltpu.SemaphoreType.DMA,
            pltpu.SemaphoreType.DMA,
            pltpu.SemaphoreType.DMA,
        ],
    )(attr2, vids, tt)
    return jnp.transpose(out2, (1, 0, 2))
